# baseline (speedup 1.0000x reference)
import jax
import jax.numpy as jnp
from jax.experimental import pallas as pl

N = 4096
BLK = 256
GRID = N // BLK
H1P = 256


def _dot(a, b):
    return jnp.dot(a, b, preferred_element_type=jnp.float32)


def _k1(x_ref, df_ref, wg1_ref, wae1_ref, bae1_ref, wae2_ref, bae2_ref,
        wd1_ref, bd1_ref, wd2_ref, bd2_ref, wa1_ref, ba1_ref, wa2_ref, ba2_ref,
        p_ref, aez_ref, aefts_ref, h2a_ref):
    p_ref[...] = _dot(df_ref[...], wg1_ref[...])
    h1 = jnp.maximum(_dot(x_ref[...], wae1_ref[...]) + bae1_ref[...], 0.0)
    z = _dot(h1, wae2_ref[...]) + bae2_ref[...]
    aez_ref[...] = z
    dh = jnp.maximum(_dot(z, wd1_ref[...]) + bd1_ref[...], 0.0)
    aefts_ref[...] = _dot(dh, wd2_ref[...]) + bd2_ref[...]
    ah = jnp.maximum(_dot(z, wa1_ref[...]) + ba1_ref[...], 0.0)
    h2a_ref[...] = _dot(ah, wa2_ref[...]) + ba2_ref[...]


def _k2(adj_ref, p_ref, bg1_ref, wg2_ref, q_ref):
    gh = jnp.maximum(_dot(adj_ref[...], p_ref[...]) + bg1_ref[...], 0.0)
    q_ref[...] = _dot(gh, wg2_ref[...])


def _k3(adj_ref, q_ref, bg2_ref,
        wd1_ref, bd1_ref, wd2_ref, bd2_ref, wa1_ref, ba1_ref, wa2_ref, ba2_ref,
        gz_ref, gfts_ref, h2g_ref):
    z = _dot(adj_ref[...], q_ref[...]) + bg2_ref[...]
    gz_ref[...] = z
    dh = jnp.maximum(_dot(z, wd1_ref[...]) + bd1_ref[...], 0.0)
    gfts_ref[...] = _dot(dh, wd2_ref[...]) + bd2_ref[...]
    ah = jnp.maximum(_dot(z, wa1_ref[...]) + ba1_ref[...], 0.0)
    h2g_ref[...] = _dot(ah, wa2_ref[...]) + ba2_ref[...]


def _k4(h2a_ref, h2g_ref, aeadj_ref, gadj_ref):
    i = pl.program_id(0)
    dn = (((1,), (1,)), ((), ()))
    blk_a = h2a_ref[pl.ds(i * BLK, BLK), :]
    aeadj_ref[...] = jax.lax.dot_general(
        blk_a, h2a_ref[...], dn, preferred_element_type=jnp.float32)
    blk_g = h2g_ref[pl.ds(i * BLK, BLK), :]
    gadj_ref[...] = jax.lax.dot_general(
        blk_g, h2g_ref[...], dn, preferred_element_type=jnp.float32)


def _full(shape):
    nd = len(shape)
    return pl.BlockSpec(shape, lambda i: (0,) * nd)


def _rows(cols):
    return pl.BlockSpec((BLK, cols), lambda i: (i, 0))


def kernel(x, adj, diag_fts, W_ae1, b_ae1, W_ae2, b_ae2, W_g1, b_g1, W_g2,
           b_g2, W_d1, b_d1, W_d2, b_d2, W_a1, b_a1, W_a2, b_a2):
    f32 = jnp.float32
    padc = lambda w: jnp.pad(w, ((0, 0), (0, H1P - w.shape[1])))
    padr = lambda w: jnp.pad(w, ((0, H1P - w.shape[0]), (0, 0)))
    padb = lambda b: jnp.pad(b, (0, H1P - b.shape[0])).reshape(1, H1P)
    row = lambda b: b.reshape(1, -1)

    W_g1p, b_g1p, W_g2p = padc(W_g1), padb(b_g1), padr(W_g2)
    W_ae1p, b_ae1p, W_ae2p = padc(W_ae1), padb(b_ae1), padr(W_ae2)
    W_d1p, b_d1p, W_d2p = padc(W_d1), padb(b_d1), padr(W_d2)

    NF = x.shape[1]
    NH = W_ae2.shape[1]

    P, ae_z, ae_fts, h2a = pl.pallas_call(
        _k1,
        grid=(GRID,),
        in_specs=[_rows(NF), _rows(N), _full((N, H1P)),
                  _full((NF, H1P)), _full((1, H1P)), _full((H1P, NH)),
                  _full((1, NH)),
                  _full((NH, H1P)), _full((1, H1P)), _full((H1P, NF)),
                  _full((1, NF)),
                  _full((NH, NH)), _full((1, NH)), _full((NH, NH)),
                  _full((1, NH))],
        out_specs=[_rows(H1P), _rows(NH), _rows(NF), _rows(NH)],
        out_shape=[jax.ShapeDtypeStruct((N, H1P), f32),
                   jax.ShapeDtypeStruct((N, NH), f32),
                   jax.ShapeDtypeStruct((N, NF), f32),
                   jax.ShapeDtypeStruct((N, NH), f32)],
    )(x, diag_fts, W_g1p, W_ae1p, b_ae1p, W_ae2p, row(b_ae2),
      W_d1p, b_d1p, W_d2p, row(b_d2), W_a1, row(b_a1), W_a2, row(b_a2))

    Q = pl.pallas_call(
        _k2,
        grid=(GRID,),
        in_specs=[_rows(N), _full((N, H1P)), _full((1, H1P)),
                  _full((H1P, NH))],
        out_specs=_rows(NH),
        out_shape=jax.ShapeDtypeStruct((N, NH), f32),
    )(adj, P, b_g1p, W_g2p)

    gae_z, gae_fts, h2g = pl.pallas_call(
        _k3,
        grid=(GRID,),
        in_specs=[_rows(N), _full((N, NH)), _full((1, NH)),
                  _full((NH, H1P)), _full((1, H1P)), _full((H1P, NF)),
                  _full((1, NF)),
                  _full((NH, NH)), _full((1, NH)), _full((NH, NH)),
                  _full((1, NH))],
        out_specs=[_rows(NH), _rows(NF), _rows(NH)],
        out_shape=[jax.ShapeDtypeStruct((N, NH), f32),
                   jax.ShapeDtypeStruct((N, NF), f32),
                   jax.ShapeDtypeStruct((N, NH), f32)],
    )(adj, Q, row(b_g2), W_d1p, b_d1p, W_d2p, row(b_d2),
      W_a1, row(b_a1), W_a2, row(b_a2))

    ae_adj, gae_adj = pl.pallas_call(
        _k4,
        grid=(GRID,),
        in_specs=[_full((N, NH)), _full((N, NH))],
        out_specs=[_rows(N), _rows(N)],
        out_shape=[jax.ShapeDtypeStruct((N, N), f32),
                   jax.ShapeDtypeStruct((N, N), f32)],
    )(h2a, h2g)

    return (ae_z, ae_fts, ae_adj, gae_z, gae_fts, gae_adj)


# speedup vs baseline: 1.0074x; 1.0074x over previous
import jax
import jax.numpy as jnp
from jax.experimental import pallas as pl

N = 4096
BLK = 256
GRID = N // BLK
H1P = 256


def _dot(a, b):
    return jnp.dot(a, b, preferred_element_type=jnp.float32)


def _bdot(a, b):
    return jnp.dot(a.astype(jnp.bfloat16), b.astype(jnp.bfloat16),
                   preferred_element_type=jnp.float32)


def _k1(x_ref, df_ref, wg1_ref, wae1_ref, bae1_ref, wae2_ref, bae2_ref,
        wd1_ref, bd1_ref, wd2_ref, bd2_ref, wa1_ref, ba1_ref, wa2_ref, ba2_ref,
        p_ref, aez_ref, aefts_ref, h2a_ref):
    p_ref[...] = _bdot(df_ref[...], wg1_ref[...])
    h1 = jnp.maximum(_dot(x_ref[...], wae1_ref[...]) + bae1_ref[...], 0.0)
    z = _dot(h1, wae2_ref[...]) + bae2_ref[...]
    aez_ref[...] = z
    dh = jnp.maximum(_dot(z, wd1_ref[...]) + bd1_ref[...], 0.0)
    aefts_ref[...] = _dot(dh, wd2_ref[...]) + bd2_ref[...]
    ah = jnp.maximum(_dot(z, wa1_ref[...]) + ba1_ref[...], 0.0)
    h2a_ref[...] = _dot(ah, wa2_ref[...]) + ba2_ref[...]


def _k2(adj_ref, p_ref, bg1_ref, wg2_ref, q_ref):
    gh = jnp.maximum(_bdot(adj_ref[...], p_ref[...]) + bg1_ref[...], 0.0)
    q_ref[...] = _dot(gh, wg2_ref[...])


def _k3(adj_ref, q_ref, bg2_ref,
        wd1_ref, bd1_ref, wd2_ref, bd2_ref, wa1_ref, ba1_ref, wa2_ref, ba2_ref,
        gz_ref, gfts_ref, h2g_ref):
    z = _bdot(adj_ref[...], q_ref[...]) + bg2_ref[...]
    gz_ref[...] = z
    dh = jnp.maximum(_dot(z, wd1_ref[...]) + bd1_ref[...], 0.0)
    gfts_ref[...] = _dot(dh, wd2_ref[...]) + bd2_ref[...]
    ah = jnp.maximum(_dot(z, wa1_ref[...]) + ba1_ref[...], 0.0)
    h2g_ref[...] = _dot(ah, wa2_ref[...]) + ba2_ref[...]


def _k4(h2a_ref, h2g_ref, aeadj_ref, gadj_ref):
    i = pl.program_id(0)
    dn = (((1,), (1,)), ((), ()))
    aeadj_ref[...] = jax.lax.dot_general(
        h2a_ref[pl.ds(i * BLK, BLK), :].astype(jnp.bfloat16),
        h2a_ref[...].astype(jnp.bfloat16), dn,
        preferred_element_type=jnp.float32)
    gadj_ref[...] = jax.lax.dot_general(
        h2g_ref[pl.ds(i * BLK, BLK), :].astype(jnp.bfloat16),
        h2g_ref[...].astype(jnp.bfloat16), dn,
        preferred_element_type=jnp.float32)


def _full(shape):
    nd = len(shape)
    return pl.BlockSpec(shape, lambda i: (0,) * nd)


def _rows(cols):
    return pl.BlockSpec((BLK, cols), lambda i: (i, 0))


def kernel(x, adj, diag_fts, W_ae1, b_ae1, W_ae2, b_ae2, W_g1, b_g1, W_g2,
           b_g2, W_d1, b_d1, W_d2, b_d2, W_a1, b_a1, W_a2, b_a2):
    f32 = jnp.float32
    padc = lambda w: jnp.pad(w, ((0, 0), (0, H1P - w.shape[1])))
    padr = lambda w: jnp.pad(w, ((0, H1P - w.shape[0]), (0, 0)))
    padb = lambda b: jnp.pad(b, (0, H1P - b.shape[0])).reshape(1, H1P)
    row = lambda b: b.reshape(1, -1)

    W_g1p, b_g1p, W_g2p = padc(W_g1), padb(b_g1), padr(W_g2)
    W_ae1p, b_ae1p, W_ae2p = padc(W_ae1), padb(b_ae1), padr(W_ae2)
    W_d1p, b_d1p, W_d2p = padc(W_d1), padb(b_d1), padr(W_d2)

    NF = x.shape[1]
    NH = W_ae2.shape[1]

    P, ae_z, ae_fts, h2a = pl.pallas_call(
        _k1,
        grid=(GRID,),
        in_specs=[_rows(NF), _rows(N), _full((N, H1P)),
                  _full((NF, H1P)), _full((1, H1P)), _full((H1P, NH)),
                  _full((1, NH)),
                  _full((NH, H1P)), _full((1, H1P)), _full((H1P, NF)),
                  _full((1, NF)),
                  _full((NH, NH)), _full((1, NH)), _full((NH, NH)),
                  _full((1, NH))],
        out_specs=[_rows(H1P), _rows(NH), _rows(NF), _rows(NH)],
        out_shape=[jax.ShapeDtypeStruct((N, H1P), f32),
                   jax.ShapeDtypeStruct((N, NH), f32),
                   jax.ShapeDtypeStruct((N, NF), f32),
                   jax.ShapeDtypeStruct((N, NH), f32)],
    )(x, diag_fts, W_g1p, W_ae1p, b_ae1p, W_ae2p, row(b_ae2),
      W_d1p, b_d1p, W_d2p, row(b_d2), W_a1, row(b_a1), W_a2, row(b_a2))

    Q = pl.pallas_call(
        _k2,
        grid=(GRID,),
        in_specs=[_rows(N), _full((N, H1P)), _full((1, H1P)),
                  _full((H1P, NH))],
        out_specs=_rows(NH),
        out_shape=jax.ShapeDtypeStruct((N, NH), f32),
    )(adj, P, b_g1p, W_g2p)

    gae_z, gae_fts, h2g = pl.pallas_call(
        _k3,
        grid=(GRID,),
        in_specs=[_rows(N), _full((N, NH)), _full((1, NH)),
                  _full((NH, H1P)), _full((1, H1P)), _full((H1P, NF)),
                  _full((1, NF)),
                  _full((NH, NH)), _full((1, NH)), _full((NH, NH)),
                  _full((1, NH))],
        out_specs=[_rows(NH), _rows(NF), _rows(NH)],
        out_shape=[jax.ShapeDtypeStruct((N, NH), f32),
                   jax.ShapeDtypeStruct((N, NF), f32),
                   jax.ShapeDtypeStruct((N, NH), f32)],
    )(adj, Q, row(b_g2), W_d1p, b_d1p, W_d2p, row(b_d2),
      W_a1, row(b_a1), W_a2, row(b_a2))

    ae_adj, gae_adj = pl.pallas_call(
        _k4,
        grid=(GRID,),
        in_specs=[_full((N, NH)), _full((N, NH))],
        out_specs=[_rows(N), _rows(N)],
        out_shape=[jax.ShapeDtypeStruct((N, N), f32),
                   jax.ShapeDtypeStruct((N, N), f32)],
    )(h2a, h2g)

    return (ae_z, ae_fts, ae_adj, gae_z, gae_fts, gae_adj)


# speedup vs baseline: 1.1134x; 1.1052x over previous
import jax
import jax.numpy as jnp
from jax.experimental import pallas as pl
from jax.experimental.pallas import tpu as pltpu

N = 4096
BLK = 256
GRID = N // BLK
H1P = 256
BF = jnp.bfloat16


def _dot(a, b):
    return jnp.dot(a, b, preferred_element_type=jnp.float32)


def _kA(x_ref, df_ref, wg1_ref, wae1_ref, bae1_ref, wae2_ref, bae2_ref,
        wd1_ref, bd1_ref, wd2_ref, bd2_ref, wa1_ref, ba1_ref, wa2_ref, ba2_ref,
        p_ref, aez_ref, aefts_ref, h2a_ref):
    p_ref[...] = _dot(df_ref[...].astype(BF), wg1_ref[...]).astype(BF)
    h1 = jnp.maximum(_dot(x_ref[...], wae1_ref[...]) + bae1_ref[...], 0.0)
    z = _dot(h1, wae2_ref[...]) + bae2_ref[...]
    aez_ref[...] = z
    dh = jnp.maximum(_dot(z, wd1_ref[...]) + bd1_ref[...], 0.0)
    aefts_ref[...] = _dot(dh, wd2_ref[...]) + bd2_ref[...]
    ah = jnp.maximum(_dot(z, wa1_ref[...]) + ba1_ref[...], 0.0)
    h2a_ref[...] = (_dot(ah, wa2_ref[...]) + ba2_ref[...]).astype(BF)


def _kB(adj_ref, p_ref, bg1_ref, wg2_ref, bg2_ref,
        wd1_ref, bd1_ref, wd2_ref, bd2_ref, wa1_ref, ba1_ref, wa2_ref, ba2_ref,
        gz_ref, gfts_ref, h2g_ref, adjb_ref, qb_ref):
    i = pl.program_id(0)

    @pl.when(i < GRID)
    def _phase1():
        ab = adj_ref[...].astype(BF)
        adjb_ref[pl.ds(i * BLK, BLK), :] = ab
        gh = jnp.maximum(_dot(ab, p_ref[...]) + bg1_ref[...], 0.0)
        qb_ref[pl.ds(i * BLK, BLK), :] = _dot(gh, wg2_ref[...]).astype(BF)

    @pl.when(i >= GRID)
    def _phase2():
        j = i - GRID
        ab = adjb_ref[pl.ds(j * BLK, BLK), :]
        z = _dot(ab, qb_ref[...]) + bg2_ref[...]
        gz_ref[...] = z
        dh = jnp.maximum(_dot(z, wd1_ref[...]) + bd1_ref[...], 0.0)
        gfts_ref[...] = _dot(dh, wd2_ref[...]) + bd2_ref[...]
        ah = jnp.maximum(_dot(z, wa1_ref[...]) + ba1_ref[...], 0.0)
        h2g_ref[...] = (_dot(ah, wa2_ref[...]) + ba2_ref[...]).astype(BF)


def _kC(h2a_ref, h2g_ref, aeadj_ref, gadj_ref):
    i = pl.program_id(0)
    dn = (((1,), (1,)), ((), ()))
    aeadj_ref[...] = jax.lax.dot_general(
        h2a_ref[pl.ds(i * BLK, BLK), :], h2a_ref[...], dn,
        preferred_element_type=jnp.float32)
    gadj_ref[...] = jax.lax.dot_general(
        h2g_ref[pl.ds(i * BLK, BLK), :], h2g_ref[...], dn,
        preferred_element_type=jnp.float32)


def _full(shape):
    nd = len(shape)
    return pl.BlockSpec(shape, lambda i: (0,) * nd)


def _rows(cols):
    return pl.BlockSpec((BLK, cols), lambda i: (i, 0))


def kernel(x, adj, diag_fts, W_ae1, b_ae1, W_ae2, b_ae2, W_g1, b_g1, W_g2,
           b_g2, W_d1, b_d1, W_d2, b_d2, W_a1, b_a1, W_a2, b_a2):
    f32 = jnp.float32
    padc = lambda w: jnp.pad(w, ((0, 0), (0, H1P - w.shape[1])))
    padr = lambda w: jnp.pad(w, ((0, H1P - w.shape[0]), (0, 0)))
    padb = lambda b: jnp.pad(b, (0, H1P - b.shape[0])).reshape(1, H1P)
    row = lambda b: b.reshape(1, -1)

    W_g1p, b_g1p, W_g2p = padc(W_g1).astype(BF), padb(b_g1), padr(W_g2)
    W_ae1p, b_ae1p, W_ae2p = padc(W_ae1), padb(b_ae1), padr(W_ae2)
    W_d1p, b_d1p, W_d2p = padc(W_d1), padb(b_d1), padr(W_d2)

    NF = x.shape[1]
    NH = W_ae2.shape[1]

    P, ae_z, ae_fts, h2a = pl.pallas_call(
        _kA,
        grid=(GRID,),
        in_specs=[_rows(NF), _rows(N), _full((N, H1P)),
                  _full((NF, H1P)), _full((1, H1P)), _full((H1P, NH)),
                  _full((1, NH)),
                  _full((NH, H1P)), _full((1, H1P)), _full((H1P, NF)),
                  _full((1, NF)),
                  _full((NH, NH)), _full((1, NH)), _full((NH, NH)),
                  _full((1, NH))],
        out_specs=[_rows(H1P), _rows(NH), _rows(NF), _rows(NH)],
        out_shape=[jax.ShapeDtypeStruct((N, H1P), BF),
                   jax.ShapeDtypeStruct((N, NH), f32),
                   jax.ShapeDtypeStruct((N, NF), f32),
                   jax.ShapeDtypeStruct((N, NH), BF)],
    )(x, diag_fts, W_g1p, W_ae1p, b_ae1p, W_ae2p, row(b_ae2),
      W_d1p, b_d1p, W_d2p, row(b_d2), W_a1, row(b_a1), W_a2, row(b_a2))

    adj_spec = pl.BlockSpec(
        (BLK, N), lambda i: (jnp.where(i < GRID, i, GRID - 1), 0))
    out_rows = lambda cols: pl.BlockSpec(
        (BLK, cols), lambda i: (jnp.where(i < GRID, 0, i - GRID), 0))

    gae_z, gae_fts, h2g = pl.pallas_call(
        _kB,
        grid=(2 * GRID,),
        in_specs=[adj_spec, _full((N, H1P)), _full((1, H1P)),
                  _full((H1P, NH)), _full((1, NH)),
                  _full((NH, H1P)), _full((1, H1P)), _full((H1P, NF)),
                  _full((1, NF)),
                  _full((NH, NH)), _full((1, NH)), _full((NH, NH)),
                  _full((1, NH))],
        out_specs=[out_rows(NH), out_rows(NF), out_rows(NH)],
        out_shape=[jax.ShapeDtypeStruct((N, NH), f32),
                   jax.ShapeDtypeStruct((N, NF), f32),
                   jax.ShapeDtypeStruct((N, NH), BF)],
        scratch_shapes=[pltpu.VMEM((N, N), BF), pltpu.VMEM((N, NH), BF)],
    )(adj, P, b_g1p, W_g2p, row(b_g2), W_d1p, b_d1p, W_d2p, row(b_d2),
      W_a1, row(b_a1), W_a2, row(b_a2))

    ae_adj, gae_adj = pl.pallas_call(
        _kC,
        grid=(GRID,),
        in_specs=[_full((N, NH)), _full((N, NH))],
        out_specs=[_rows(N), _rows(N)],
        out_shape=[jax.ShapeDtypeStruct((N, N), f32),
                   jax.ShapeDtypeStruct((N, N), f32)],
    )(h2a, h2g)

    return (ae_z, ae_fts, ae_adj, gae_z, gae_fts, gae_adj)


# speedup vs baseline: 1.1845x; 1.0639x over previous
import jax
import jax.numpy as jnp
from jax.experimental import pallas as pl
from jax.experimental.pallas import tpu as pltpu

N = 4096
BLK = 512
GRID = N // BLK
H1P = 256
BF = jnp.bfloat16


def _dot(a, b):
    return jnp.dot(a, b, preferred_element_type=jnp.float32)


def _kA(x_ref, df_ref, wg1_ref, wae1_ref, bae1_ref, wae2_ref, bae2_ref,
        wd1_ref, bd1_ref, wd2_ref, bd2_ref, wa1_ref, ba1_ref, wa2_ref, ba2_ref,
        p_ref, aez_ref, aefts_ref, h2a_ref):
    p_ref[...] = _dot(df_ref[...].astype(BF), wg1_ref[...]).astype(BF)
    h1 = jnp.maximum(_dot(x_ref[...], wae1_ref[...]) + bae1_ref[...], 0.0)
    z = _dot(h1, wae2_ref[...]) + bae2_ref[...]
    aez_ref[...] = z
    dh = jnp.maximum(_dot(z, wd1_ref[...]) + bd1_ref[...], 0.0)
    aefts_ref[...] = _dot(dh, wd2_ref[...]) + bd2_ref[...]
    ah = jnp.maximum(_dot(z, wa1_ref[...]) + ba1_ref[...], 0.0)
    h2a_ref[...] = (_dot(ah, wa2_ref[...]) + ba2_ref[...]).astype(BF)


def _kB(adj_ref, p_ref, bg1_ref, wg2_ref, bg2_ref,
        wd1_ref, bd1_ref, wd2_ref, bd2_ref, wa1_ref, ba1_ref, wa2_ref, ba2_ref,
        gz_ref, gfts_ref, h2g_ref, adjb_ref, qb_ref):
    i = pl.program_id(0)

    @pl.when(i < GRID)
    def _phase1():
        ab = adj_ref[...].astype(BF)
        adjb_ref[pl.ds(i * BLK, BLK), :] = ab
        gh = jnp.maximum(_dot(ab, p_ref[...]) + bg1_ref[...], 0.0)
        qb_ref[pl.ds(i * BLK, BLK), :] = _dot(gh, wg2_ref[...]).astype(BF)

    @pl.when(i >= GRID)
    def _phase2():
        j = i - GRID
        ab = adjb_ref[pl.ds(j * BLK, BLK), :]
        z = _dot(ab, qb_ref[...]) + bg2_ref[...]
        gz_ref[...] = z
        dh = jnp.maximum(_dot(z, wd1_ref[...]) + bd1_ref[...], 0.0)
        gfts_ref[...] = _dot(dh, wd2_ref[...]) + bd2_ref[...]
        ah = jnp.maximum(_dot(z, wa1_ref[...]) + ba1_ref[...], 0.0)
        h2g_ref[...] = (_dot(ah, wa2_ref[...]) + ba2_ref[...]).astype(BF)


def _kC(h2a_ref, h2g_ref, aeadj_ref, gadj_ref):
    i = pl.program_id(0)
    dn = (((1,), (1,)), ((), ()))
    aeadj_ref[...] = jax.lax.dot_general(
        h2a_ref[pl.ds(i * BLK, BLK), :], h2a_ref[...], dn,
        preferred_element_type=jnp.float32)
    gadj_ref[...] = jax.lax.dot_general(
        h2g_ref[pl.ds(i * BLK, BLK), :], h2g_ref[...], dn,
        preferred_element_type=jnp.float32)


def _full(shape):
    nd = len(shape)
    return pl.BlockSpec(shape, lambda i: (0,) * nd)


def _rows(cols):
    return pl.BlockSpec((BLK, cols), lambda i: (i, 0))


def kernel(x, adj, diag_fts, W_ae1, b_ae1, W_ae2, b_ae2, W_g1, b_g1, W_g2,
           b_g2, W_d1, b_d1, W_d2, b_d2, W_a1, b_a1, W_a2, b_a2):
    f32 = jnp.float32
    padc = lambda w: jnp.pad(w, ((0, 0), (0, H1P - w.shape[1])))
    padr = lambda w: jnp.pad(w, ((0, H1P - w.shape[0]), (0, 0)))
    padb = lambda b: jnp.pad(b, (0, H1P - b.shape[0])).reshape(1, H1P)
    row = lambda b: b.reshape(1, -1)

    W_g1p, b_g1p, W_g2p = padc(W_g1).astype(BF), padb(b_g1), padr(W_g2)
    W_ae1p, b_ae1p, W_ae2p = padc(W_ae1), padb(b_ae1), padr(W_ae2)
    W_d1p, b_d1p, W_d2p = padc(W_d1), padb(b_d1), padr(W_d2)

    NF = x.shape[1]
    NH = W_ae2.shape[1]

    P, ae_z, ae_fts, h2a = pl.pallas_call(
        _kA,
        grid=(GRID,),
        in_specs=[_rows(NF), _rows(N), _full((N, H1P)),
                  _full((NF, H1P)), _full((1, H1P)), _full((H1P, NH)),
                  _full((1, NH)),
                  _full((NH, H1P)), _full((1, H1P)), _full((H1P, NF)),
                  _full((1, NF)),
                  _full((NH, NH)), _full((1, NH)), _full((NH, NH)),
                  _full((1, NH))],
        out_specs=[_rows(H1P), _rows(NH), _rows(NF), _rows(NH)],
        out_shape=[jax.ShapeDtypeStruct((N, H1P), BF),
                   jax.ShapeDtypeStruct((N, NH), f32),
                   jax.ShapeDtypeStruct((N, NF), f32),
                   jax.ShapeDtypeStruct((N, NH), BF)],
    )(x, diag_fts, W_g1p, W_ae1p, b_ae1p, W_ae2p, row(b_ae2),
      W_d1p, b_d1p, W_d2p, row(b_d2), W_a1, row(b_a1), W_a2, row(b_a2))

    adj_spec = pl.BlockSpec(
        (BLK, N), lambda i: (jnp.where(i < GRID, i, GRID - 1), 0))
    out_rows = lambda cols: pl.BlockSpec(
        (BLK, cols), lambda i: (jnp.where(i < GRID, 0, i - GRID), 0))

    gae_z, gae_fts, h2g = pl.pallas_call(
        _kB,
        grid=(2 * GRID,),
        in_specs=[adj_spec, _full((N, H1P)), _full((1, H1P)),
                  _full((H1P, NH)), _full((1, NH)),
                  _full((NH, H1P)), _full((1, H1P)), _full((H1P, NF)),
                  _full((1, NF)),
                  _full((NH, NH)), _full((1, NH)), _full((NH, NH)),
                  _full((1, NH))],
        out_specs=[out_rows(NH), out_rows(NF), out_rows(NH)],
        out_shape=[jax.ShapeDtypeStruct((N, NH), f32),
                   jax.ShapeDtypeStruct((N, NF), f32),
                   jax.ShapeDtypeStruct((N, NH), BF)],
        scratch_shapes=[pltpu.VMEM((N, N), BF), pltpu.VMEM((N, NH), BF)],
    )(adj, P, b_g1p, W_g2p, row(b_g2), W_d1p, b_d1p, W_d2p, row(b_d2),
      W_a1, row(b_a1), W_a2, row(b_a2))

    ae_adj, gae_adj = pl.pallas_call(
        _kC,
        grid=(GRID,),
        in_specs=[_full((N, NH)), _full((N, NH))],
        out_specs=[_rows(N), _rows(N)],
        out_shape=[jax.ShapeDtypeStruct((N, N), f32),
                   jax.ShapeDtypeStruct((N, N), f32)],
    )(h2a, h2g)

    return (ae_z, ae_fts, ae_adj, gae_z, gae_fts, gae_adj)


# speedup vs baseline: 1.2033x; 1.0159x over previous
import jax
import jax.numpy as jnp
from jax.experimental import pallas as pl
from jax.experimental.pallas import tpu as pltpu

N = 4096
H1P = 256
BF = jnp.bfloat16
BLKA = 512
GRIDA = N // BLKA
BLKB = 256
GRIDB = N // BLKB
BLKC = 512
GRIDC = N // BLKC


def _dot(a, b):
    return jnp.dot(a, b, preferred_element_type=jnp.float32)


def _kA(x_ref, df_ref, wg1_ref, wae1_ref, bae1_ref, wae2_ref, bae2_ref,
        wd1_ref, bd1_ref, wd2_ref, bd2_ref, wa1_ref, ba1_ref, wa2_ref, ba2_ref,
        p_ref, aez_ref, aefts_ref, h2a_ref):
    p_ref[...] = _dot(df_ref[...].astype(BF), wg1_ref[...]).astype(BF)
    h1 = jnp.maximum(_dot(x_ref[...], wae1_ref[...]) + bae1_ref[...], 0.0)
    z = _dot(h1, wae2_ref[...]) + bae2_ref[...]
    aez_ref[...] = z
    dh = jnp.maximum(_dot(z, wd1_ref[...]) + bd1_ref[...], 0.0)
    aefts_ref[...] = _dot(dh, wd2_ref[...]) + bd2_ref[...]
    ah = jnp.maximum(_dot(z, wa1_ref[...]) + ba1_ref[...], 0.0)
    h2a_ref[...] = (_dot(ah, wa2_ref[...]) + ba2_ref[...]).astype(BF)


def _kB(adj_ref, p_ref, bg1_ref, wg2_ref, bg2_ref, h2a_ref,
        wd1_ref, bd1_ref, wd2_ref, bd2_ref, wa1_ref, ba1_ref, wa2_ref, ba2_ref,
        gz_ref, gfts_ref, h2g_ref, aeadj_ref, adjb_ref, qb_ref):
    i = pl.program_id(0)

    @pl.when(i < GRIDB)
    def _phase1():
        ab = adj_ref[...].astype(BF)
        adjb_ref[pl.ds(i * BLKB, BLKB), :] = ab
        gh = jnp.maximum(_dot(ab, p_ref[...]) + bg1_ref[...], 0.0)
        qb_ref[pl.ds(i * BLKB, BLKB), :] = _dot(gh, wg2_ref[...]).astype(BF)

    @pl.when(i >= GRIDB)
    def _phase2():
        j = i - GRIDB
        ab = adjb_ref[pl.ds(j * BLKB, BLKB), :]
        z = _dot(ab, qb_ref[...]) + bg2_ref[...]
        gz_ref[...] = z
        dh = jnp.maximum(_dot(z, wd1_ref[...]) + bd1_ref[...], 0.0)
        gfts_ref[...] = _dot(dh, wd2_ref[...]) + bd2_ref[...]
        ah = jnp.maximum(_dot(z, wa1_ref[...]) + ba1_ref[...], 0.0)
        h2g_ref[...] = (_dot(ah, wa2_ref[...]) + ba2_ref[...]).astype(BF)
        dn = (((1,), (1,)), ((), ()))
        aeadj_ref[...] = jax.lax.dot_general(
            h2a_ref[pl.ds(j * BLKB, BLKB), :], h2a_ref[...], dn,
            preferred_element_type=jnp.float32)


def _kC(h2g_ref, gadj_ref):
    i = pl.program_id(0)
    dn = (((1,), (1,)), ((), ()))
    gadj_ref[...] = jax.lax.dot_general(
        h2g_ref[pl.ds(i * BLKC, BLKC), :], h2g_ref[...], dn,
        preferred_element_type=jnp.float32)


def _full(shape):
    nd = len(shape)
    return pl.BlockSpec(shape, lambda i: (0,) * nd)


def _rows(blk, cols):
    return pl.BlockSpec((blk, cols), lambda i: (i, 0))


def kernel(x, adj, diag_fts, W_ae1, b_ae1, W_ae2, b_ae2, W_g1, b_g1, W_g2,
           b_g2, W_d1, b_d1, W_d2, b_d2, W_a1, b_a1, W_a2, b_a2):
    f32 = jnp.float32
    padc = lambda w: jnp.pad(w, ((0, 0), (0, H1P - w.shape[1])))
    padr = lambda w: jnp.pad(w, ((0, H1P - w.shape[0]), (0, 0)))
    padb = lambda b: jnp.pad(b, (0, H1P - b.shape[0])).reshape(1, H1P)
    row = lambda b: b.reshape(1, -1)

    W_g1p, b_g1p, W_g2p = padc(W_g1).astype(BF), padb(b_g1), padr(W_g2)
    W_ae1p, b_ae1p, W_ae2p = padc(W_ae1), padb(b_ae1), padr(W_ae2)
    W_d1p, b_d1p, W_d2p = padc(W_d1), padb(b_d1), padr(W_d2)

    NF = x.shape[1]
    NH = W_ae2.shape[1]

    P, ae_z, ae_fts, h2a = pl.pallas_call(
        _kA,
        grid=(GRIDA,),
        in_specs=[_rows(BLKA, NF), _rows(BLKA, N), _full((N, H1P)),
                  _full((NF, H1P)), _full((1, H1P)), _full((H1P, NH)),
                  _full((1, NH)),
                  _full((NH, H1P)), _full((1, H1P)), _full((H1P, NF)),
                  _full((1, NF)),
                  _full((NH, NH)), _full((1, NH)), _full((NH, NH)),
                  _full((1, NH))],
        out_specs=[_rows(BLKA, H1P), _rows(BLKA, NH), _rows(BLKA, NF),
                   _rows(BLKA, NH)],
        out_shape=[jax.ShapeDtypeStruct((N, H1P), BF),
                   jax.ShapeDtypeStruct((N, NH), f32),
                   jax.ShapeDtypeStruct((N, NF), f32),
                   jax.ShapeDtypeStruct((N, NH), BF)],
    )(x, diag_fts, W_g1p, W_ae1p, b_ae1p, W_ae2p, row(b_ae2),
      W_d1p, b_d1p, W_d2p, row(b_d2), W_a1, row(b_a1), W_a2, row(b_a2))

    adj_spec = pl.BlockSpec(
        (BLKB, N), lambda i: (jnp.where(i < GRIDB, i, GRIDB - 1), 0))
    out_rows = lambda cols: pl.BlockSpec(
        (BLKB, cols), lambda i: (jnp.where(i < GRIDB, 0, i - GRIDB), 0))

    gae_z, gae_fts, h2g, ae_adj = pl.pallas_call(
        _kB,
        grid=(2 * GRIDB,),
        in_specs=[adj_spec, _full((N, H1P)), _full((1, H1P)),
                  _full((H1P, NH)), _full((1, NH)), _full((N, NH)),
                  _full((NH, H1P)), _full((1, H1P)), _full((H1P, NF)),
                  _full((1, NF)),
                  _full((NH, NH)), _full((1, NH)), _full((NH, NH)),
                  _full((1, NH))],
        out_specs=[out_rows(NH), out_rows(NF), out_rows(NH), out_rows(N)],
        out_shape=[jax.ShapeDtypeStruct((N, NH), f32),
                   jax.ShapeDtypeStruct((N, NF), f32),
                   jax.ShapeDtypeStruct((N, NH), BF),
                   jax.ShapeDtypeStruct((N, N), f32)],
        scratch_shapes=[pltpu.VMEM((N, N), BF), pltpu.VMEM((N, NH), BF)],
    )(adj, P, b_g1p, W_g2p, row(b_g2), h2a,
      W_d1p, b_d1p, W_d2p, row(b_d2), W_a1, row(b_a1), W_a2, row(b_a2))

    gae_adj = pl.pallas_call(
        _kC,
        grid=(GRIDC,),
        in_specs=[_full((N, NH))],
        out_specs=_rows(BLKC, N),
        out_shape=jax.ShapeDtypeStruct((N, N), f32),
    )(h2g)

    return (ae_z, ae_fts, ae_adj, gae_z, gae_fts, gae_adj)
